# baseline (speedup 1.0000x reference)
import jax
import jax.numpy as jnp
from jax.experimental import pallas as pl
import jax.experimental.pallas.tpu as pltpu

KH = 3
KW = 3
RH = 56
HALO = 8


def _conv_kernel(w_ref, m_ref, xm_ref, xh_ref, o_ref):
    f, rh, wo = o_ref.shape
    c, _, w = xm_ref.shape
    w_eff = (w_ref[...] * m_ref[...]).astype(jnp.bfloat16)
    xfull = jnp.concatenate(
        [xm_ref[...], xh_ref[...]], axis=1
    ).astype(jnp.bfloat16)
    xsh = [xfull, jnp.roll(xfull, -1, axis=2), jnp.roll(xfull, -2, axis=2)]
    xcol = jnp.concatenate(
        [xsh[j][:, i:i + rh, :].reshape(c, rh * w)
         for i in range(KH) for j in range(KW)],
        axis=0,
    )
    m = jax.lax.dot_general(
        w_eff, xcol, (((1,), (0,)), ((), ())),
        preferred_element_type=jnp.float32,
    ).reshape(f, rh, w)
    o_ref[...] = m[:, :, :wo]


def kernel(x, kernel_values, kernel_mask):
    b, c, h, w = x.shape
    f = kernel_values.shape[0]
    ho = h - KH + 1
    wo = w - KW + 1
    n_rb = h // RH
    n_hb = h // HALO
    pd = c * KH * KW

    ratio = RH // HALO

    out = pl.pallas_call(
        _conv_kernel,
        grid=(b, n_rb),
        in_specs=[
            pl.BlockSpec((f, pd), lambda bi, ri: (0, 0)),
            pl.BlockSpec((f, pd), lambda bi, ri: (0, 0)),
            pl.BlockSpec((pl.squeezed, c, RH, w), lambda bi, ri: (bi, 0, ri, 0)),
            pl.BlockSpec(
                (pl.squeezed, c, HALO, w),
                lambda bi, ri: (bi, 0, jnp.minimum(ratio * ri + ratio, n_hb - 1), 0),
            ),
        ],
        out_specs=pl.BlockSpec(
            (pl.squeezed, f, RH, wo), lambda bi, ri: (bi, 0, ri, 0)
        ),
        out_shape=jax.ShapeDtypeStruct((b, f, ho, wo), jnp.float32),
        compiler_params=pltpu.CompilerParams(
            dimension_semantics=("parallel", "arbitrary"),
        ),
    )(kernel_values, kernel_mask, x, x)
    return out


# speedup vs baseline: 1.4244x; 1.4244x over previous
import jax
import jax.numpy as jnp
from jax.experimental import pallas as pl
import jax.experimental.pallas.tpu as pltpu

KH = 3
KW = 3
RH = 32
HALO = 8


def _conv_kernel(w_ref, m_ref, xm_ref, xh_ref, o_ref):
    f, rh, wo = o_ref.shape
    c, _, w = xm_ref.shape
    w_eff = (w_ref[...] * m_ref[...]).astype(jnp.bfloat16)
    xfull = jnp.concatenate(
        [xm_ref[...], xh_ref[...]], axis=1
    ).astype(jnp.bfloat16)
    flat = xfull[:, :rh + KH - 1, :].reshape(c, (rh + KH - 1) * w)
    n = rh * w
    pieces = []
    for i in range(KH):
        for j in range(KW):
            off = w * i + j
            rolled = flat if off == 0 else jnp.roll(flat, -off, axis=1)
            pieces.append(rolled[:, :n])
    xcol = jnp.concatenate(pieces, axis=0)
    m = jax.lax.dot_general(
        w_eff, xcol, (((1,), (0,)), ((), ())),
        preferred_element_type=jnp.float32,
    ).reshape(f, rh, w)
    o_ref[...] = m[:, :, :wo]


def kernel(x, kernel_values, kernel_mask):
    b, c, h, w = x.shape
    f = kernel_values.shape[0]
    ho = h - KH + 1
    wo = w - KW + 1
    n_rb = h // RH
    n_hb = h // HALO
    pd = c * KH * KW

    ratio = RH // HALO

    out = pl.pallas_call(
        _conv_kernel,
        grid=(b, n_rb),
        in_specs=[
            pl.BlockSpec((f, pd), lambda bi, ri: (0, 0)),
            pl.BlockSpec((f, pd), lambda bi, ri: (0, 0)),
            pl.BlockSpec((pl.squeezed, c, RH, w), lambda bi, ri: (bi, 0, ri, 0)),
            pl.BlockSpec(
                (pl.squeezed, c, HALO, w),
                lambda bi, ri: (bi, 0, jnp.minimum(ratio * ri + ratio, n_hb - 1), 0),
            ),
        ],
        out_specs=pl.BlockSpec(
            (pl.squeezed, f, RH, wo), lambda bi, ri: (bi, 0, ri, 0)
        ),
        out_shape=jax.ShapeDtypeStruct((b, f, ho, wo), jnp.float32),
        compiler_params=pltpu.CompilerParams(
            dimension_semantics=("parallel", "arbitrary"),
        ),
    )(kernel_values, kernel_mask, x, x)
    return out
